# baseline (speedup 1.0000x reference)
import jax
import jax.numpy as jnp
import numpy as np
from jax import lax
from jax.experimental import pallas as pl
from jax.experimental.pallas import tpu as pltpu

_B = 2
_NP = 512
_C = 256
_NC = 81
_ROI = 7
_SR = 2
_SCALE = 1.0 / 32.0
_FH = 50
_FW = 50
_ST = 0.05
_NMS_T = 0.5
_DETS = 100
_NEG = -1e4
_CLIP = float(np.log(1000.0 / 16.0))
_FEAT = _C * _ROI * _ROI
_R = 32
_KT = 896


def _fiota(shape, dim):
    return lax.broadcasted_iota(jnp.int32, shape, dim).astype(jnp.float32)


def _interp_mat(lo, bsz, size):
    R = lo.shape[0]
    M = R * _ROI * _SR
    e_row = jnp.floor(_fiota((M, R), 0) / (_ROI * _SR))
    E = (e_row == _fiota((M, R), 1)).astype(jnp.float32)
    lob = jnp.dot(E, jnp.concatenate([lo, bsz], axis=1),
                  preferred_element_type=jnp.float32)
    lom = lob[:, 0:1]
    bszm = lob[:, 1:2]
    mi = _fiota((M, 1), 0)
    q = mi - jnp.floor(mi / (_ROI * _SR)) * (_ROI * _SR)
    i = jnp.floor(q / _SR)
    s = q - i * _SR
    pts = lom + i * bszm + (s + 0.5) * bszm / _SR
    valid = ((pts > -1.0) & (pts < size)).astype(jnp.float32)
    pc = jnp.clip(pts, 0.0, size - 1.0)
    p0 = jnp.floor(pc)
    lw = (pc - p0) * valid
    hw = (1.0 - (pc - p0)) * valid
    p1 = jnp.minimum(p0 + 1.0, size - 1.0)
    grid = _fiota((M, size), 1)
    w = jnp.where(grid == p0, hw, 0.0) + jnp.where(grid == p1, lw, 0.0)
    pair = (jnp.floor(_fiota((R * _ROI, M), 1) / _SR)
            == _fiota((R * _ROI, M), 0)).astype(jnp.float32)
    return 0.5 * jnp.dot(pair, w, preferred_element_type=jnp.float32)


def _roi_kernel(prop_ref, feat_ref, out_ref):
    pr = prop_ref[...]
    x1 = pr[:, 0:1] * _SCALE
    y1 = pr[:, 1:2] * _SCALE
    x2 = pr[:, 2:3] * _SCALE
    y2 = pr[:, 3:4] * _SCALE
    bw = jnp.maximum(x2 - x1, 1.0) / _ROI
    bh = jnp.maximum(y2 - y1, 1.0) / _ROI
    ry = _interp_mat(y1, bh, _FH)
    rx = _interp_mat(x1, bw, _FW)
    tmp = jnp.dot(ry, feat_ref[...], preferred_element_type=jnp.float32)
    t4 = tmp.reshape(_R, _ROI, _FW, _C)
    rx3 = rx.reshape(_R, _ROI, _FW)
    out = lax.dot_general(rx3, t4, (((2,), (2,)), ((0,), (0,))),
                          preferred_element_type=jnp.float32)
    out_ref[...] = out.reshape(_R, _FEAT)


def _fc1_kernel(x_ref, w_ref, b_ref, out_ref, acc_ref):
    k = pl.program_id(0)
    m = pl.program_id(1)
    row = pl.ds(m * _NP, _NP)

    @pl.when(k == 0)
    def _init():
        acc_ref[row, :] = jnp.zeros((_NP, 1024), jnp.float32)

    acc_ref[row, :] += jnp.dot(x_ref[...], w_ref[...],
                               preferred_element_type=jnp.float32)

    @pl.when(k == (_FEAT // _KT) - 1)
    def _fin():
        out_ref[...] = jnp.maximum(acc_ref[row, :] + b_ref[...], 0.0)


def _post_kernel(h1_ref, w2_ref, b2_ref, wc_ref, bc_ref, wb_ref, bb_ref,
                 prop_ref, img_ref, ob_ref, os_ref, ol_ref):
    h2 = jnp.maximum(
        jnp.dot(h1_ref[...], w2_ref[...], preferred_element_type=jnp.float32)
        + b2_ref[...], 0.0)
    logits = jnp.dot(h2, wc_ref[...], preferred_element_type=jnp.float32) + bc_ref[...]
    br = jnp.dot(h2, wb_ref[...], preferred_element_type=jnp.float32) + bb_ref[...]
    m = jnp.max(logits, axis=1, keepdims=True)
    e = jnp.exp(logits - m)
    p = e / jnp.sum(e, axis=1, keepdims=True)
    scores = p[:, 1:]
    dx = br[:, 1:81] / 10.0
    dy = br[:, 82:162] / 10.0
    dw = jnp.minimum(br[:, 163:243] / 5.0, _CLIP)
    dh = jnp.minimum(br[:, 244:324] / 5.0, _CLIP)
    prp = prop_ref[...]
    px1 = prp[:, 0:1]
    py1 = prp[:, 1:2]
    px2 = prp[:, 2:3]
    py2 = prp[:, 3:4]
    pw = px2 - px1
    ph = py2 - py1
    pcx = px1 + 0.5 * pw
    pcy = py1 + 0.5 * ph
    pcx2 = dx * pw + pcx
    pcy2 = dy * ph + pcy
    pw2 = jnp.exp(dw) * pw
    ph2 = jnp.exp(dh) * ph
    img = img_ref[...]
    hs = img[:, 0:1]
    ws = img[:, 1:2]
    x1 = jnp.clip(pcx2 - 0.5 * pw2, 0.0, ws)
    y1 = jnp.clip(pcy2 - 0.5 * ph2, 0.0, hs)
    x2 = jnp.clip(pcx2 + 0.5 * pw2, 0.0, ws)
    y2 = jnp.clip(pcy2 + 0.5 * ph2, 0.0, hs)
    s0 = jnp.where(scores > _ST, scores, _NEG)

    ci = _fiota((_NP, _NC - 1), 1)
    pi = _fiota((_NP, _NC - 1), 0)
    ki = pi * (_NC - 1) + ci
    lab = ci + 1.0
    off = lab * 2048.0
    xo1 = x1 + off
    yo1 = y1 + off
    xo2 = x2 + off
    yo2 = y2 + off
    areas = (xo2 - xo1) * (yo2 - yo1)

    ob_ref[...] = jnp.zeros((_DETS, 4), jnp.float32)
    os_ref[...] = jnp.zeros((_DETS, 1), jnp.float32)
    ol_ref[...] = jnp.zeros((_DETS, 1), jnp.int32)

    def cond(carry):
        t, _, m = carry
        return (t < _DETS) & (m > _ST)

    def body(carry):
        t, s, m = carry
        bk = jnp.min(jnp.where(s == m, ki, 1e9))
        mask = ki == bk

        def pick(v):
            return jnp.sum(jnp.where(mask, v, 0.0))

        bx1 = pick(x1)
        by1 = pick(y1)
        bx2 = pick(x2)
        by2 = pick(y2)
        bl = bk - jnp.floor(bk / (_NC - 1)) * (_NC - 1) + 1.0
        boff = bl * 2048.0
        bxo1 = bx1 + boff
        byo1 = by1 + boff
        bxo2 = bx2 + boff
        byo2 = by2 + boff
        xx1 = jnp.maximum(bxo1, xo1)
        yy1 = jnp.maximum(byo1, yo1)
        xx2 = jnp.minimum(bxo2, xo2)
        yy2 = jnp.minimum(byo2, yo2)
        inter = jnp.maximum(xx2 - xx1, 0.0) * jnp.maximum(yy2 - yy1, 0.0)
        ba = (bxo2 - bxo1) * (byo2 - byo1)
        iou = inter / (ba + areas - inter + 1e-9)
        s = jnp.where(iou > _NMS_T, _NEG, s)
        s = jnp.where(mask, _NEG, s)
        col = _fiota((1, 4), 1)
        row = jnp.where(col == 0.0, bx1,
                        jnp.where(col == 1.0, by1,
                                  jnp.where(col == 2.0, bx2, by2)))
        ob_ref[pl.ds(t, 1), :] = row
        os_ref[pl.ds(t, 1), :] = m * jnp.ones((1, 1), jnp.float32)
        ol_ref[pl.ds(t, 1), :] = bl.astype(jnp.int32) * jnp.ones((1, 1), jnp.int32)
        return t + 1, s, jnp.max(s)

    lax.while_loop(cond, body, (0, s0, jnp.max(s0)))


def kernel(features, proposals, image_shapes, W1, b1, W2, b2, Wc, bc, Wb, bb):
    feat_t = features.transpose(0, 2, 3, 1).reshape(_B, _FH, _FW * _C)

    pooled = pl.pallas_call(
        _roi_kernel,
        grid=(_B, _NP // _R),
        in_specs=[
            pl.BlockSpec((None, _R, 4), lambda b, c: (b, c, 0)),
            pl.BlockSpec((None, _FH, _FW * _C), lambda b, c: (b, 0, 0)),
        ],
        out_specs=pl.BlockSpec((None, _R, _FEAT), lambda b, c: (b, c, 0)),
        out_shape=jax.ShapeDtypeStruct((_B, _NP, _FEAT), jnp.float32),
    )(proposals, feat_t)
    x = pooled.reshape(_B * _NP, _FEAT)

    w1t = W1.reshape(1024, _C, _ROI, _ROI).transpose(3, 2, 1, 0).reshape(_FEAT, 1024)
    nk = _FEAT // _KT
    h1 = pl.pallas_call(
        _fc1_kernel,
        grid=(nk, _B),
        in_specs=[
            pl.BlockSpec((_NP, _KT), lambda k, mm: (mm, k)),
            pl.BlockSpec((_KT, 1024), lambda k, mm: (k, 0)),
            pl.BlockSpec((1, 1024), lambda k, mm: (0, 0)),
        ],
        out_specs=pl.BlockSpec((_NP, 1024), lambda k, mm: (mm, 0)),
        out_shape=jax.ShapeDtypeStruct((_B * _NP, 1024), jnp.float32),
        scratch_shapes=[pltpu.VMEM((_B * _NP, 1024), jnp.float32)],
    )(x, w1t, b1.reshape(1, 1024))

    w2t = W2.T
    wct = Wc.T
    wbt = Wb.reshape(_NC, 4, 1024).transpose(1, 0, 2).reshape(4 * _NC, 1024).T
    bbr = bb.reshape(_NC, 4).T.reshape(1, 4 * _NC)
    img_f = image_shapes.astype(jnp.float32).reshape(_B, 1, 2)

    ob, os_, ol = pl.pallas_call(
        _post_kernel,
        grid=(_B,),
        in_specs=[
            pl.BlockSpec((_NP, 1024), lambda b: (b, 0)),
            pl.BlockSpec((1024, 1024), lambda b: (0, 0)),
            pl.BlockSpec((1, 1024), lambda b: (0, 0)),
            pl.BlockSpec((1024, _NC), lambda b: (0, 0)),
            pl.BlockSpec((1, _NC), lambda b: (0, 0)),
            pl.BlockSpec((1024, 4 * _NC), lambda b: (0, 0)),
            pl.BlockSpec((1, 4 * _NC), lambda b: (0, 0)),
            pl.BlockSpec((None, _NP, 4), lambda b: (b, 0, 0)),
            pl.BlockSpec((None, 1, 2), lambda b: (b, 0, 0)),
        ],
        out_specs=[
            pl.BlockSpec((None, _DETS, 4), lambda b: (b, 0, 0)),
            pl.BlockSpec((None, _DETS, 1), lambda b: (b, 0, 0)),
            pl.BlockSpec((None, _DETS, 1), lambda b: (b, 0, 0)),
        ],
        out_shape=[
            jax.ShapeDtypeStruct((_B, _DETS, 4), jnp.float32),
            jax.ShapeDtypeStruct((_B, _DETS, 1), jnp.float32),
            jax.ShapeDtypeStruct((_B, _DETS, 1), jnp.int32),
        ],
    )(h1, w2t, b2.reshape(1, 1024), wct, bc.reshape(1, _NC), wbt, bbr,
      proposals, img_f)

    return ob, os_.reshape(_B, _DETS), ol.reshape(_B, _DETS)


# speedup vs baseline: 1.0300x; 1.0300x over previous
import jax
import jax.numpy as jnp
import numpy as np
from jax import lax
from jax.experimental import pallas as pl
from jax.experimental.pallas import tpu as pltpu

_B = 2
_NP = 512
_C = 256
_NC = 81
_ROI = 7
_SR = 2
_SCALE = 1.0 / 32.0
_FH = 50
_FW = 50
_ST = 0.05
_NMS_T = 0.5
_DETS = 100
_NEG = -1e4
_CLIP = float(np.log(1000.0 / 16.0))
_FEAT = _C * _ROI * _ROI
_R = 32
_KT = 896


def _fiota(shape, dim):
    return lax.broadcasted_iota(jnp.int32, shape, dim).astype(jnp.float32)


def _interp_mat(lo, bsz, size):
    R = lo.shape[0]
    M = R * _ROI * _SR
    e_row = jnp.floor(_fiota((M, R), 0) / (_ROI * _SR))
    E = (e_row == _fiota((M, R), 1)).astype(jnp.float32)
    lob = jnp.dot(E, jnp.concatenate([lo, bsz], axis=1),
                  preferred_element_type=jnp.float32)
    lom = lob[:, 0:1]
    bszm = lob[:, 1:2]
    mi = _fiota((M, 1), 0)
    q = mi - jnp.floor(mi / (_ROI * _SR)) * (_ROI * _SR)
    i = jnp.floor(q / _SR)
    s = q - i * _SR
    pts = lom + i * bszm + (s + 0.5) * bszm / _SR
    valid = ((pts > -1.0) & (pts < size)).astype(jnp.float32)
    pc = jnp.clip(pts, 0.0, size - 1.0)
    p0 = jnp.floor(pc)
    lw = (pc - p0) * valid
    hw = (1.0 - (pc - p0)) * valid
    p1 = jnp.minimum(p0 + 1.0, size - 1.0)
    grid = _fiota((M, size), 1)
    w = jnp.where(grid == p0, hw, 0.0) + jnp.where(grid == p1, lw, 0.0)
    pair = (jnp.floor(_fiota((R * _ROI, M), 1) / _SR)
            == _fiota((R * _ROI, M), 0)).astype(jnp.float32)
    return 0.5 * jnp.dot(pair, w, preferred_element_type=jnp.float32)


def _roi_kernel(prop_ref, feat_ref, out_ref):
    pr = prop_ref[...]
    x1 = pr[:, 0:1] * _SCALE
    y1 = pr[:, 1:2] * _SCALE
    x2 = pr[:, 2:3] * _SCALE
    y2 = pr[:, 3:4] * _SCALE
    bw = jnp.maximum(x2 - x1, 1.0) / _ROI
    bh = jnp.maximum(y2 - y1, 1.0) / _ROI
    ry = _interp_mat(y1, bh, _FH)
    rx = _interp_mat(x1, bw, _FW)
    tmp = jnp.dot(ry, feat_ref[...], preferred_element_type=jnp.float32)
    t4 = tmp.reshape(_R, _ROI, _FW, _C)
    rx3 = rx.reshape(_R, _ROI, _FW)
    out = lax.dot_general(rx3, t4, (((2,), (2,)), ((0,), (0,))),
                          preferred_element_type=jnp.float32)
    out_ref[...] = out.reshape(_R, _FEAT)


def _fc1_kernel(x_ref, w_ref, b_ref, out_ref, acc_ref):
    k = pl.program_id(1)

    @pl.when(k == 0)
    def _init():
        acc_ref[...] = jnp.zeros_like(acc_ref)

    acc_ref[...] += jnp.dot(x_ref[...], w_ref[...],
                            preferred_element_type=jnp.float32)

    @pl.when(k == (_FEAT // _KT) - 1)
    def _fin():
        out_ref[...] = jnp.maximum(acc_ref[...] + b_ref[...], 0.0)


def _post_kernel(h1_ref, w2_ref, b2_ref, wc_ref, bc_ref, wb_ref, bb_ref,
                 prop_ref, img_ref, ob_ref, os_ref, ol_ref):
    h2 = jnp.maximum(
        jnp.dot(h1_ref[...], w2_ref[...], preferred_element_type=jnp.float32)
        + b2_ref[...], 0.0)
    logits = jnp.dot(h2, wc_ref[...], preferred_element_type=jnp.float32) + bc_ref[...]
    br = jnp.dot(h2, wb_ref[...], preferred_element_type=jnp.float32) + bb_ref[...]
    m = jnp.max(logits, axis=1, keepdims=True)
    e = jnp.exp(logits - m)
    p = e / jnp.sum(e, axis=1, keepdims=True)
    scores = p[:, 1:]
    dx = br[:, 1:81] / 10.0
    dy = br[:, 82:162] / 10.0
    dw = jnp.minimum(br[:, 163:243] / 5.0, _CLIP)
    dh = jnp.minimum(br[:, 244:324] / 5.0, _CLIP)
    prp = prop_ref[...]
    px1 = prp[:, 0:1]
    py1 = prp[:, 1:2]
    px2 = prp[:, 2:3]
    py2 = prp[:, 3:4]
    pw = px2 - px1
    ph = py2 - py1
    pcx = px1 + 0.5 * pw
    pcy = py1 + 0.5 * ph
    pcx2 = dx * pw + pcx
    pcy2 = dy * ph + pcy
    pw2 = jnp.exp(dw) * pw
    ph2 = jnp.exp(dh) * ph
    img = img_ref[...]
    hs = img[:, 0:1]
    ws = img[:, 1:2]
    x1 = jnp.clip(pcx2 - 0.5 * pw2, 0.0, ws)
    y1 = jnp.clip(pcy2 - 0.5 * ph2, 0.0, hs)
    x2 = jnp.clip(pcx2 + 0.5 * pw2, 0.0, ws)
    y2 = jnp.clip(pcy2 + 0.5 * ph2, 0.0, hs)
    s0 = jnp.where(scores > _ST, scores, _NEG)

    ci = _fiota((_NP, _NC - 1), 1)
    pi = _fiota((_NP, _NC - 1), 0)
    ki = pi * (_NC - 1) + ci
    lab = ci + 1.0
    off = lab * 2048.0
    xo1 = x1 + off
    yo1 = y1 + off
    xo2 = x2 + off
    yo2 = y2 + off
    areas = (xo2 - xo1) * (yo2 - yo1)

    ob_ref[...] = jnp.zeros((_DETS, 4), jnp.float32)
    os_ref[...] = jnp.zeros((_DETS, 1), jnp.float32)
    ol_ref[...] = jnp.zeros((_DETS, 1), jnp.int32)

    def cond(carry):
        t, _, m = carry
        return (t < _DETS) & (m > _ST)

    def body(carry):
        t, s, m = carry
        bk = jnp.min(jnp.where(s == m, ki, 1e9))
        mask = ki == bk

        def pick(v):
            return jnp.sum(jnp.where(mask, v, 0.0))

        bx1 = pick(x1)
        by1 = pick(y1)
        bx2 = pick(x2)
        by2 = pick(y2)
        bl = bk - jnp.floor(bk / (_NC - 1)) * (_NC - 1) + 1.0
        boff = bl * 2048.0
        bxo1 = bx1 + boff
        byo1 = by1 + boff
        bxo2 = bx2 + boff
        byo2 = by2 + boff
        xx1 = jnp.maximum(bxo1, xo1)
        yy1 = jnp.maximum(byo1, yo1)
        xx2 = jnp.minimum(bxo2, xo2)
        yy2 = jnp.minimum(byo2, yo2)
        inter = jnp.maximum(xx2 - xx1, 0.0) * jnp.maximum(yy2 - yy1, 0.0)
        ba = (bxo2 - bxo1) * (byo2 - byo1)
        iou = inter / (ba + areas - inter + 1e-9)
        s = jnp.where(iou > _NMS_T, _NEG, s)
        s = jnp.where(mask, _NEG, s)
        col = _fiota((1, 4), 1)
        row = jnp.where(col == 0.0, bx1,
                        jnp.where(col == 1.0, by1,
                                  jnp.where(col == 2.0, bx2, by2)))
        ob_ref[pl.ds(t, 1), :] = row
        os_ref[pl.ds(t, 1), :] = m * jnp.ones((1, 1), jnp.float32)
        ol_ref[pl.ds(t, 1), :] = bl.astype(jnp.int32) * jnp.ones((1, 1), jnp.int32)
        return t + 1, s, jnp.max(s)

    lax.while_loop(cond, body, (0, s0, jnp.max(s0)))


def kernel(features, proposals, image_shapes, W1, b1, W2, b2, Wc, bc, Wb, bb):
    feat_t = features.transpose(0, 2, 3, 1).reshape(_B, _FH, _FW * _C)

    pooled = pl.pallas_call(
        _roi_kernel,
        grid=(_B, _NP // _R),
        in_specs=[
            pl.BlockSpec((None, _R, 4), lambda b, c: (b, c, 0)),
            pl.BlockSpec((None, _FH, _FW * _C), lambda b, c: (b, 0, 0)),
        ],
        out_specs=pl.BlockSpec((None, _R, _FEAT), lambda b, c: (b, c, 0)),
        out_shape=jax.ShapeDtypeStruct((_B, _NP, _FEAT), jnp.float32),
    )(proposals, feat_t)
    x = pooled.reshape(_B * _NP, _FEAT)

    w1t = W1.reshape(1024, _C, _ROI, _ROI).transpose(3, 2, 1, 0).reshape(_FEAT, 1024)
    nk = _FEAT // _KT
    h1 = pl.pallas_call(
        _fc1_kernel,
        grid=(_B, nk),
        in_specs=[
            pl.BlockSpec((_NP, _KT), lambda mm, k: (mm, k)),
            pl.BlockSpec((_KT, 1024), lambda mm, k: (k, 0)),
            pl.BlockSpec((1, 1024), lambda mm, k: (0, 0)),
        ],
        out_specs=pl.BlockSpec((_NP, 1024), lambda mm, k: (mm, 0)),
        out_shape=jax.ShapeDtypeStruct((_B * _NP, 1024), jnp.float32),
        scratch_shapes=[pltpu.VMEM((_NP, 1024), jnp.float32)],
    )(x, w1t, b1.reshape(1, 1024))

    w2t = W2.T
    wct = Wc.T
    wbt = Wb.reshape(_NC, 4, 1024).transpose(1, 0, 2).reshape(4 * _NC, 1024).T
    bbr = bb.reshape(_NC, 4).T.reshape(1, 4 * _NC)
    img_f = image_shapes.astype(jnp.float32).reshape(_B, 1, 2)

    ob, os_, ol = pl.pallas_call(
        _post_kernel,
        grid=(_B,),
        in_specs=[
            pl.BlockSpec((_NP, 1024), lambda b: (b, 0)),
            pl.BlockSpec((1024, 1024), lambda b: (0, 0)),
            pl.BlockSpec((1, 1024), lambda b: (0, 0)),
            pl.BlockSpec((1024, _NC), lambda b: (0, 0)),
            pl.BlockSpec((1, _NC), lambda b: (0, 0)),
            pl.BlockSpec((1024, 4 * _NC), lambda b: (0, 0)),
            pl.BlockSpec((1, 4 * _NC), lambda b: (0, 0)),
            pl.BlockSpec((None, _NP, 4), lambda b: (b, 0, 0)),
            pl.BlockSpec((None, 1, 2), lambda b: (b, 0, 0)),
        ],
        out_specs=[
            pl.BlockSpec((None, _DETS, 4), lambda b: (b, 0, 0)),
            pl.BlockSpec((None, _DETS, 1), lambda b: (b, 0, 0)),
            pl.BlockSpec((None, _DETS, 1), lambda b: (b, 0, 0)),
        ],
        out_shape=[
            jax.ShapeDtypeStruct((_B, _DETS, 4), jnp.float32),
            jax.ShapeDtypeStruct((_B, _DETS, 1), jnp.float32),
            jax.ShapeDtypeStruct((_B, _DETS, 1), jnp.int32),
        ],
    )(h1, w2t, b2.reshape(1, 1024), wct, bc.reshape(1, _NC), wbt, bbr,
      proposals, img_f)

    return ob, os_.reshape(_B, _DETS), ol.reshape(_B, _DETS)


# speedup vs baseline: 1.0662x; 1.0351x over previous
import jax
import jax.numpy as jnp
import numpy as np
from jax import lax
from jax.experimental import pallas as pl
from jax.experimental.pallas import tpu as pltpu

_B = 2
_NP = 512
_C = 256
_NC = 81
_ROI = 7
_SR = 2
_SCALE = 1.0 / 32.0
_FH = 50
_FW = 50
_ST = 0.05
_NMS_T = 0.5
_DETS = 100
_NEG = -1e4
_CLIP = float(np.log(1000.0 / 16.0))
_FEAT = _C * _ROI * _ROI
_R = 32
_KT = 896


def _fiota(shape, dim):
    return lax.broadcasted_iota(jnp.int32, shape, dim).astype(jnp.float32)


def _interp_mat(lo, bsz, size):
    R = lo.shape[0]
    M = R * _ROI * _SR
    e_row = jnp.floor(_fiota((M, R), 0) / (_ROI * _SR))
    E = (e_row == _fiota((M, R), 1)).astype(jnp.float32)
    lob = jnp.dot(E, jnp.concatenate([lo, bsz], axis=1),
                  preferred_element_type=jnp.float32)
    lom = lob[:, 0:1]
    bszm = lob[:, 1:2]
    mi = _fiota((M, 1), 0)
    q = mi - jnp.floor(mi / (_ROI * _SR)) * (_ROI * _SR)
    i = jnp.floor(q / _SR)
    s = q - i * _SR
    pts = lom + i * bszm + (s + 0.5) * bszm / _SR
    valid = ((pts > -1.0) & (pts < size)).astype(jnp.float32)
    pc = jnp.clip(pts, 0.0, size - 1.0)
    p0 = jnp.floor(pc)
    lw = (pc - p0) * valid
    hw = (1.0 - (pc - p0)) * valid
    p1 = jnp.minimum(p0 + 1.0, size - 1.0)
    grid = _fiota((M, size), 1)
    w = jnp.where(grid == p0, hw, 0.0) + jnp.where(grid == p1, lw, 0.0)
    pair = (jnp.floor(_fiota((R * _ROI, M), 1) / _SR)
            == _fiota((R * _ROI, M), 0)).astype(jnp.float32)
    return 0.5 * jnp.dot(pair, w, preferred_element_type=jnp.float32)


def _roi_kernel(prop_ref, feat_ref, out_ref):
    pr = prop_ref[...]
    x1 = pr[:, 0:1] * _SCALE
    y1 = pr[:, 1:2] * _SCALE
    x2 = pr[:, 2:3] * _SCALE
    y2 = pr[:, 3:4] * _SCALE
    bw = jnp.maximum(x2 - x1, 1.0) / _ROI
    bh = jnp.maximum(y2 - y1, 1.0) / _ROI
    ry = _interp_mat(y1, bh, _FH)
    rx = _interp_mat(x1, bw, _FW)
    tmp = jnp.dot(ry, feat_ref[...], preferred_element_type=jnp.float32)
    t4 = tmp.reshape(_R, _ROI, _FW, _C)
    rx3 = rx.reshape(_R, _ROI, _FW)
    out = lax.dot_general(rx3, t4, (((2,), (2,)), ((0,), (0,))),
                          preferred_element_type=jnp.float32)
    out_ref[...] = out.reshape(_R, _FEAT)


def _fc1_kernel(x_ref, w_ref, b_ref, out_ref, acc_ref):
    k = pl.program_id(0)

    @pl.when(k == 0)
    def _init():
        acc_ref[...] = jnp.zeros_like(acc_ref)

    acc_ref[...] += jnp.dot(x_ref[...], w_ref[...],
                            preferred_element_type=jnp.float32)

    @pl.when(k == (_FEAT // _KT) - 1)
    def _fin():
        out_ref[...] = jnp.maximum(acc_ref[...] + b_ref[...], 0.0)


def _post_kernel(h1_ref, w2_ref, b2_ref, wc_ref, bc_ref, wb_ref, bb_ref,
                 prop_ref, img_ref, ob_ref, os_ref, ol_ref):
    h2 = jnp.maximum(
        jnp.dot(h1_ref[...], w2_ref[...], preferred_element_type=jnp.float32)
        + b2_ref[...], 0.0)
    logits = jnp.dot(h2, wc_ref[...], preferred_element_type=jnp.float32) + bc_ref[...]
    br = jnp.dot(h2, wb_ref[...], preferred_element_type=jnp.float32) + bb_ref[...]
    m = jnp.max(logits, axis=1, keepdims=True)
    e = jnp.exp(logits - m)
    p = e / jnp.sum(e, axis=1, keepdims=True)
    scores = p[:, 1:]
    dx = br[:, 1:81] / 10.0
    dy = br[:, 82:162] / 10.0
    dw = jnp.minimum(br[:, 163:243] / 5.0, _CLIP)
    dh = jnp.minimum(br[:, 244:324] / 5.0, _CLIP)
    prp = prop_ref[...]
    px1 = prp[:, 0:1]
    py1 = prp[:, 1:2]
    px2 = prp[:, 2:3]
    py2 = prp[:, 3:4]
    pw = px2 - px1
    ph = py2 - py1
    pcx = px1 + 0.5 * pw
    pcy = py1 + 0.5 * ph
    pcx2 = dx * pw + pcx
    pcy2 = dy * ph + pcy
    pw2 = jnp.exp(dw) * pw
    ph2 = jnp.exp(dh) * ph
    img = img_ref[...]
    hs = img[:, 0:1]
    ws = img[:, 1:2]
    x1 = jnp.clip(pcx2 - 0.5 * pw2, 0.0, ws)
    y1 = jnp.clip(pcy2 - 0.5 * ph2, 0.0, hs)
    x2 = jnp.clip(pcx2 + 0.5 * pw2, 0.0, ws)
    y2 = jnp.clip(pcy2 + 0.5 * ph2, 0.0, hs)
    s0 = jnp.where(scores > _ST, scores, _NEG)

    ci = _fiota((_NP, _NC - 1), 1)
    pi = _fiota((_NP, _NC - 1), 0)
    ki = pi * (_NC - 1) + ci
    lab = ci + 1.0
    off = lab * 2048.0
    xo1 = x1 + off
    yo1 = y1 + off
    xo2 = x2 + off
    yo2 = y2 + off
    areas = (xo2 - xo1) * (yo2 - yo1)

    ob_ref[...] = jnp.zeros((_DETS, 4), jnp.float32)
    os_ref[...] = jnp.zeros((_DETS, 1), jnp.float32)
    ol_ref[...] = jnp.zeros((_DETS, 1), jnp.int32)

    def cond(carry):
        t, _, m = carry
        return (t < _DETS) & (m > _ST)

    def body(carry):
        t, s, m = carry
        bk = jnp.min(jnp.where(s == m, ki, 1e9))
        mask = ki == bk

        def pick(v):
            return jnp.sum(jnp.where(mask, v, 0.0))

        bx1 = pick(x1)
        by1 = pick(y1)
        bx2 = pick(x2)
        by2 = pick(y2)
        bl = bk - jnp.floor(bk / (_NC - 1)) * (_NC - 1) + 1.0
        boff = bl * 2048.0
        bxo1 = bx1 + boff
        byo1 = by1 + boff
        bxo2 = bx2 + boff
        byo2 = by2 + boff
        xx1 = jnp.maximum(bxo1, xo1)
        yy1 = jnp.maximum(byo1, yo1)
        xx2 = jnp.minimum(bxo2, xo2)
        yy2 = jnp.minimum(byo2, yo2)
        inter = jnp.maximum(xx2 - xx1, 0.0) * jnp.maximum(yy2 - yy1, 0.0)
        ba = (bxo2 - bxo1) * (byo2 - byo1)
        iou = inter / (ba + areas - inter + 1e-9)
        s = jnp.where(iou > _NMS_T, _NEG, s)
        s = jnp.where(mask, _NEG, s)
        col = _fiota((1, 4), 1)
        row = jnp.where(col == 0.0, bx1,
                        jnp.where(col == 1.0, by1,
                                  jnp.where(col == 2.0, bx2, by2)))
        ob_ref[pl.ds(t, 1), :] = row
        os_ref[pl.ds(t, 1), :] = m * jnp.ones((1, 1), jnp.float32)
        ol_ref[pl.ds(t, 1), :] = bl.astype(jnp.int32) * jnp.ones((1, 1), jnp.int32)
        return t + 1, s, jnp.max(s)

    lax.while_loop(cond, body, (0, s0, jnp.max(s0)))


def kernel(features, proposals, image_shapes, W1, b1, W2, b2, Wc, bc, Wb, bb):
    feat_t = features.transpose(0, 2, 3, 1).reshape(_B, _FH, _FW * _C)

    pooled = pl.pallas_call(
        _roi_kernel,
        grid=(_B, _NP // _R),
        in_specs=[
            pl.BlockSpec((None, _R, 4), lambda b, c: (b, c, 0)),
            pl.BlockSpec((None, _FH, _FW * _C), lambda b, c: (b, 0, 0)),
        ],
        out_specs=pl.BlockSpec((None, _R, _FEAT), lambda b, c: (b, c, 0)),
        out_shape=jax.ShapeDtypeStruct((_B, _NP, _FEAT), jnp.float32),
    )(proposals, feat_t)
    x = pooled.reshape(_B * _NP, _FEAT)

    w1t = W1.reshape(1024, _C, _ROI, _ROI).transpose(3, 2, 1, 0).reshape(_FEAT, 1024)
    nk = _FEAT // _KT
    h1 = pl.pallas_call(
        _fc1_kernel,
        grid=(nk,),
        in_specs=[
            pl.BlockSpec((_B * _NP, _KT), lambda k: (0, k)),
            pl.BlockSpec((_KT, 1024), lambda k: (k, 0)),
            pl.BlockSpec((1, 1024), lambda k: (0, 0)),
        ],
        out_specs=pl.BlockSpec((_B * _NP, 1024), lambda k: (0, 0)),
        out_shape=jax.ShapeDtypeStruct((_B * _NP, 1024), jnp.float32),
        scratch_shapes=[pltpu.VMEM((_B * _NP, 1024), jnp.float32)],
    )(x, w1t, b1.reshape(1, 1024))

    w2t = W2.T
    wct = Wc.T
    wbt = Wb.reshape(_NC, 4, 1024).transpose(1, 0, 2).reshape(4 * _NC, 1024).T
    bbr = bb.reshape(_NC, 4).T.reshape(1, 4 * _NC)
    img_f = image_shapes.astype(jnp.float32).reshape(_B, 1, 2)

    ob, os_, ol = pl.pallas_call(
        _post_kernel,
        grid=(_B,),
        in_specs=[
            pl.BlockSpec((_NP, 1024), lambda b: (b, 0)),
            pl.BlockSpec((1024, 1024), lambda b: (0, 0)),
            pl.BlockSpec((1, 1024), lambda b: (0, 0)),
            pl.BlockSpec((1024, _NC), lambda b: (0, 0)),
            pl.BlockSpec((1, _NC), lambda b: (0, 0)),
            pl.BlockSpec((1024, 4 * _NC), lambda b: (0, 0)),
            pl.BlockSpec((1, 4 * _NC), lambda b: (0, 0)),
            pl.BlockSpec((None, _NP, 4), lambda b: (b, 0, 0)),
            pl.BlockSpec((None, 1, 2), lambda b: (b, 0, 0)),
        ],
        out_specs=[
            pl.BlockSpec((None, _DETS, 4), lambda b: (b, 0, 0)),
            pl.BlockSpec((None, _DETS, 1), lambda b: (b, 0, 0)),
            pl.BlockSpec((None, _DETS, 1), lambda b: (b, 0, 0)),
        ],
        out_shape=[
            jax.ShapeDtypeStruct((_B, _DETS, 4), jnp.float32),
            jax.ShapeDtypeStruct((_B, _DETS, 1), jnp.float32),
            jax.ShapeDtypeStruct((_B, _DETS, 1), jnp.int32),
        ],
    )(h1, w2t, b2.reshape(1, 1024), wct, bc.reshape(1, _NC), wbt, bbr,
      proposals, img_f)

    return ob, os_.reshape(_B, _DETS), ol.reshape(_B, _DETS)
